# baseline (speedup 1.0000x reference)
import functools
import jax
import jax.numpy as jnp
from jax import lax
from jax.experimental import pallas as pl
from jax.experimental.pallas import tpu as pltpu
from jax.experimental.pallas import tpu_sc as plsc

NUM_C = 150
HBINS = 22528
STRIP = HBINS // 16
N_IMG, N_ROW, N_COL = 16, 512, 512
SLAB = 16
CHUNK = SLAB * N_COL
NCHUNK = (N_ROW // 2) // SLAB
NBUF = 4

_mesh = plsc.VectorSubcoreMesh(core_axis_name="c", subcore_axis_name="s")


@functools.partial(
    pl.kernel,
    mesh=_mesh,
    compiler_params=pltpu.CompilerParams(needs_layout_passes=False),
    out_type=jax.ShapeDtypeStruct((2, HBINS), jnp.float32),
    scratch_types=(
        [pltpu.VMEM((SLAB, N_COL), jnp.int32)] * (2 * NBUF)
        + [
            pltpu.VMEM((HBINS,), jnp.float32),
            pltpu.VMEM((STRIP,), jnp.float32),
            pltpu.VMEM((STRIP,), jnp.float32),
            pltpu.VMEM((STRIP,), jnp.float32),
            pltpu.VMEM((STRIP,), jnp.float32),
            pltpu.VMEM_SHARED((16, HBINS), jnp.float32),
        ]
        + [pltpu.SemaphoreType.DMA] * (2 * NBUF)
    ),
)
def _conf_hist(pred_hbm, tgt_hbm, zeros_hbm, out_hbm, *scratch):
    ring = scratch[: 2 * NBUF]
    hist_v = scratch[2 * NBUF]
    strips = scratch[2 * NBUF + 1: 2 * NBUF + 5]
    shared = scratch[2 * NBUF + 5]
    sems = scratch[2 * NBUF + 6:]

    cid = lax.axis_index("c")
    sid = lax.axis_index("s")
    wid = sid * 2 + cid
    img = wid // 2
    row0 = (wid % 2) * (N_ROW // 2)

    bufs = tuple(
        (ring[2 * b], ring[2 * b + 1], sems[2 * b], sems[2 * b + 1])
        for b in range(NBUF)
    )

    def start(r0, b):
        pbuf, tbuf, sp, st = bufs[b]
        pltpu.async_copy(pred_hbm.at[img, pl.ds(r0, SLAB), :], pbuf, sp)
        pltpu.async_copy(tgt_hbm.at[img, pl.ds(r0, SLAB), :], tbuf, st)

    def wait(b):
        pbuf, tbuf, sp, st = bufs[b]
        pltpu.make_async_copy(pred_hbm.at[0, pl.ds(0, SLAB), :], pbuf, sp).wait()
        pltpu.make_async_copy(tgt_hbm.at[0, pl.ds(0, SLAB), :], tbuf, st).wait()

    ones = jnp.ones((16,), jnp.float32)

    def compute(b):
        pbuf, tbuf, _, _ = bufs[b]

        @plsc.parallel_loop(0, CHUNK // 16, unroll=8)
        def _(j):
            r = lax.shift_right_logical(j, 5)
            c = lax.bitwise_and(j, 31) * 16
            p = pbuf[r, pl.ds(c, 16)]
            t = tbuf[r, pl.ds(c, 16)]
            idx = t * NUM_C + p
            plsc.addupdate_scatter(hist_v, [idx], ones)

    for b in range(NBUF):
        start(row0 + b * SLAB, b)
    pltpu.sync_copy(zeros_hbm, hist_v)

    def chunk_body(i, _):
        c0 = NBUF * i
        for b in range(NBUF):
            c = c0 + b
            wait(b)
            compute(b)
            nxt = c + NBUF
            rn = jnp.where(nxt < NCHUNK, row0 + nxt * SLAB, 0)
            start(rn, b)
        return 0

    lax.fori_loop(0, NCHUNK // NBUF, chunk_body, 0)
    for b in range(NBUF):
        wait(b)

    pltpu.sync_copy(hist_v, shared.at[sid])
    plsc.subcore_barrier()

    col = sid * STRIP
    sbufs = tuple((strips[b], sems[b]) for b in range(4))

    def strip_start(k, b):
        buf, sem = sbufs[b]
        pltpu.async_copy(shared.at[k, pl.ds(col, STRIP)], buf, sem)

    def strip_wait(b):
        buf, sem = sbufs[b]
        pltpu.make_async_copy(shared.at[0, pl.ds(col, STRIP)], buf, sem).wait()

    for k in range(4):
        strip_start(k, k)
    for k in range(16):
        b = k & 3
        buf, _ = sbufs[b]
        strip_wait(b)
        if k == 0:
            @plsc.parallel_loop(0, STRIP // 16, unroll=8)
            def _(m, _buf=buf):
                s = pl.ds(col + m * 16, 16)
                hist_v[s] = _buf[pl.ds(m * 16, 16)]
        else:
            @plsc.parallel_loop(0, STRIP // 16, unroll=8)
            def _(m, _buf=buf):
                s = pl.ds(col + m * 16, 16)
                hist_v[s] = hist_v[s] + _buf[pl.ds(m * 16, 16)]
        if k + 4 < 16:
            strip_start(k + 4, b)

    pltpu.sync_copy(hist_v.at[pl.ds(col, STRIP)],
                    out_hbm.at[cid, pl.ds(col, STRIP)])


def kernel(input, target):
    pred = input.astype(jnp.int32)
    tgt = target.astype(jnp.int32)
    zeros = jnp.zeros((HBINS,), jnp.float32)
    parts = _conf_hist(pred, tgt, zeros)
    return (parts[0] + parts[1])[: NUM_C * NUM_C].reshape(NUM_C, NUM_C)


# speedup vs baseline: 1.0376x; 1.0376x over previous
import functools
import jax
import jax.numpy as jnp
from jax import lax
from jax.experimental import pallas as pl
from jax.experimental.pallas import tpu as pltpu
from jax.experimental.pallas import tpu_sc as plsc

NUM_C = 150
HBINS = 22528
STRIP = HBINS // 16
N_IMG, N_ROW, N_COL = 16, 512, 512
SLAB = 32
CHUNK = SLAB * N_COL
NCHUNK = (N_ROW // 2) // SLAB

_mesh = plsc.VectorSubcoreMesh(core_axis_name="c", subcore_axis_name="s")


@functools.partial(
    pl.kernel,
    mesh=_mesh,
    compiler_params=pltpu.CompilerParams(needs_layout_passes=False),
    out_type=jax.ShapeDtypeStruct((2, HBINS), jnp.float32),
    scratch_types=[
        pltpu.VMEM((SLAB, N_COL), jnp.int32),
        pltpu.VMEM((SLAB, N_COL), jnp.int32),
        pltpu.VMEM((SLAB, N_COL), jnp.int32),
        pltpu.VMEM((SLAB, N_COL), jnp.int32),
        pltpu.VMEM((HBINS,), jnp.float32),
        pltpu.VMEM((STRIP,), jnp.float32),
        pltpu.VMEM((STRIP,), jnp.float32),
        pltpu.VMEM((STRIP,), jnp.float32),
        pltpu.VMEM((STRIP,), jnp.float32),
        pltpu.VMEM_SHARED((16, HBINS), jnp.float32),
        pltpu.SemaphoreType.DMA,
        pltpu.SemaphoreType.DMA,
        pltpu.SemaphoreType.DMA,
        pltpu.SemaphoreType.DMA,
    ],
)
def _conf_hist(pred_hbm, tgt_hbm, zeros_hbm, out_hbm,
               pred0_v, tgt0_v, pred1_v, tgt1_v, hist_v,
               strip0_v, strip1_v, strip2_v, strip3_v,
               shared, sp0, st0, sp1, st1):
    cid = lax.axis_index("c")
    sid = lax.axis_index("s")
    wid = sid * 2 + cid
    img = wid // 2
    row0 = (wid % 2) * (N_ROW // 2)

    bufs = ((pred0_v, tgt0_v, sp0, st0), (pred1_v, tgt1_v, sp1, st1))

    def start(r0, b):
        pbuf, tbuf, sp, st = bufs[b]
        pltpu.async_copy(pred_hbm.at[img, pl.ds(r0, SLAB), :], pbuf, sp)
        pltpu.async_copy(tgt_hbm.at[img, pl.ds(r0, SLAB), :], tbuf, st)

    def wait(b):
        pbuf, tbuf, sp, st = bufs[b]
        pltpu.make_async_copy(pred_hbm.at[0, pl.ds(0, SLAB), :], pbuf, sp).wait()
        pltpu.make_async_copy(tgt_hbm.at[0, pl.ds(0, SLAB), :], tbuf, st).wait()

    ones = jnp.ones((16,), jnp.float32)

    def compute(b):
        pbuf, tbuf, _, _ = bufs[b]

        @plsc.parallel_loop(0, CHUNK // 16, unroll=8)
        def _(j):
            r = lax.shift_right_logical(j, 5)
            c = lax.bitwise_and(j, 31) * 16
            p = pbuf[r, pl.ds(c, 16)]
            t = tbuf[r, pl.ds(c, 16)]
            idx = t * NUM_C + p
            plsc.addupdate_scatter(hist_v, [idx], ones)

    start(row0, 0)
    pltpu.sync_copy(zeros_hbm, hist_v)

    def chunk_body(i, _):
        c0 = 2 * i
        start(row0 + (c0 + 1) * SLAB, 1)
        wait(0)
        compute(0)
        nxt = c0 + 2
        rn = jnp.where(nxt < NCHUNK, row0 + nxt * SLAB, 0)
        start(rn, 0)
        wait(1)
        compute(1)
        return 0

    lax.fori_loop(0, NCHUNK // 2, chunk_body, 0)
    wait(0)

    pltpu.sync_copy(hist_v, shared.at[sid])
    plsc.subcore_barrier()

    col = sid * STRIP
    sbufs = ((strip0_v, sp0), (strip1_v, st0), (strip2_v, sp1),
             (strip3_v, st1))

    def strip_start(k, b):
        buf, sem = sbufs[b]
        pltpu.async_copy(shared.at[k, pl.ds(col, STRIP)], buf, sem)

    def strip_wait(b):
        buf, sem = sbufs[b]
        pltpu.make_async_copy(shared.at[0, pl.ds(col, STRIP)], buf, sem).wait()

    for k in range(4):
        strip_start(k, k)
    for k in range(16):
        b = k & 3
        buf, _ = sbufs[b]
        strip_wait(b)
        if k == 0:
            @plsc.parallel_loop(0, STRIP // 16, unroll=8)
            def _(m, _buf=buf):
                s = pl.ds(col + m * 16, 16)
                hist_v[s] = _buf[pl.ds(m * 16, 16)]
        else:
            @plsc.parallel_loop(0, STRIP // 16, unroll=8)
            def _(m, _buf=buf):
                s = pl.ds(col + m * 16, 16)
                hist_v[s] = hist_v[s] + _buf[pl.ds(m * 16, 16)]
        if k + 4 < 16:
            strip_start(k + 4, b)

    pltpu.sync_copy(hist_v.at[pl.ds(col, STRIP)],
                    out_hbm.at[cid, pl.ds(col, STRIP)])


def kernel(input, target):
    pred = input.astype(jnp.int32)
    tgt = target.astype(jnp.int32)
    zeros = jnp.zeros((HBINS,), jnp.float32)
    parts = _conf_hist(pred, tgt, zeros)
    return (parts[0] + parts[1])[: NUM_C * NUM_C].reshape(NUM_C, NUM_C)
